# baseline (speedup 1.0000x reference)
import functools

import jax
import jax.numpy as jnp
from jax import lax
from jax.experimental import pallas as pl
from jax.experimental.pallas import tpu as pltpu
from jax.experimental.pallas import tpu_sc as plsc

N = 10000
NPAD = 10240
E = 320000
EPAD = 327680
HID = 128
C3 = 256
HW3 = 14
NTILES = 32
VPT = NPAD // NTILES
EPT = EPAD // NTILES
ECH = 64
NCH = EPT // ECH
ROWS_PT = NPAD // 16

_mesh = plsc.VectorSubcoreMesh(core_axis_name="c", subcore_axis_name="s",
                               num_cores=2, num_subcores=16)


def _prep_body(vx_ref, vy_ref, idx_ref, w_ref):
    m = float(HW3 - 1)
    px = (vx_ref[...] + 1.0) * 0.5 * m
    py = (vy_ref[...] + 1.0) * 0.5 * m
    x0f = jnp.floor(px)
    y0f = jnp.floor(py)
    wx = px - x0f
    wy = py - y0f
    x0 = jnp.clip(x0f, 0.0, m).astype(jnp.int32)
    x1 = jnp.clip(x0f + 1.0, 0.0, m).astype(jnp.int32)
    y0 = jnp.clip(y0f, 0.0, m).astype(jnp.int32)
    y1 = jnp.clip(y0f + 1.0, 0.0, m).astype(jnp.int32)
    idx_ref[0] = y0 * HW3 + x0
    idx_ref[1] = y0 * HW3 + x1
    idx_ref[2] = y1 * HW3 + x0
    idx_ref[3] = y1 * HW3 + x1
    w_ref[0] = (1.0 - wx) * (1.0 - wy)
    w_ref[1] = wx * (1.0 - wy)
    w_ref[2] = (1.0 - wx) * wy
    w_ref[3] = wx * wy


_prep = pl.pallas_call(
    _prep_body,
    out_shape=(
        jax.ShapeDtypeStruct((4, 80, 128), jnp.int32),
        jax.ShapeDtypeStruct((4, 80, 128), jnp.float32),
    ),
)


@functools.partial(
    pl.kernel,
    out_type=jax.ShapeDtypeStruct((4, NPAD, C3), jnp.float32),
    mesh=_mesh,
    scratch_types=[
        pltpu.VMEM((64,), jnp.int32),
        pltpu.VMEM((64, C3), jnp.float32),
        pltpu.SemaphoreType.DMA,
    ],
)
def _vgather(tab_hbm, idx_hbm, out_hbm, idxbuf, rowsbuf, sem):
    cid = lax.axis_index("c")
    sid = lax.axis_index("s")
    wid = cid * 16 + sid
    base = wid * VPT
    for q in range(4):
        for t in range(VPT // 64):
            off = base + t * 64
            pltpu.sync_copy(idx_hbm.at[q, pl.ds(off, 64)], idxbuf)
            pltpu.async_copy(tab_hbm.at[idxbuf], rowsbuf, sem).wait()
            pltpu.sync_copy(rowsbuf, out_hbm.at[q, pl.ds(off, 64)])


def _interp_body(r_ref, w_ref, o_ref):
    acc = r_ref[0] * w_ref[0][:, :, None]
    for q in range(1, 4):
        acc = acc + r_ref[q] * w_ref[q][:, :, None]
    o_ref[...] = acc


_interp = pl.pallas_call(
    _interp_body,
    grid=(10,),
    in_specs=[
        pl.BlockSpec((4, 8, 128, C3), lambda g: (0, g, 0, 0)),
        pl.BlockSpec((4, 8, 128), lambda g: (0, g, 0)),
    ],
    out_specs=pl.BlockSpec((8, 128, C3), lambda g: (g, 0, 0)),
    out_shape=jax.ShapeDtypeStruct((80, 128, C3), jnp.float32),
)


NBUF = 4
QCH = NCH // 4


@functools.partial(
    pl.kernel,
    out_type=jax.ShapeDtypeStruct((2, NPAD, HID), jnp.float32),
    mesh=_mesh,
    scratch_types=[
        pltpu.VMEM((QCH, ECH), jnp.int32),
        pltpu.VMEM((QCH, ECH), jnp.int32),
        pltpu.VMEM((NBUF, ECH, HID), jnp.float32),
        pltpu.VMEM_SHARED((NPAD, HID), jnp.float32),
        pltpu.SemaphoreType.DMA((NBUF,)),
        pltpu.SemaphoreType.DMA((NBUF,)),
        pltpu.SemaphoreType.DMA,
    ],
)
def _agg(x_hbm, src_hbm, dst_hbm, out_hbm, sidx, didx, rows, aggsh,
         gsem, ssem, isem):
    cid = lax.axis_index("c")
    sid = lax.axis_index("s")
    wid = cid * 16 + sid

    def _load_idx(q):
        icp = pltpu.async_copy(src_hbm.at[wid, pl.ds(q * QCH, QCH)], sidx,
                               isem)
        icp2 = pltpu.async_copy(dst_hbm.at[wid, pl.ds(q * QCH, QCH)], didx,
                                isem)
        icp.wait()
        icp2.wait()

    zv = jnp.zeros((16,), jnp.float32)

    def _zrow(i, carry):
        for j in range(HID // 16):
            rows[0, i, pl.ds(j * 16, 16)] = zv
        return carry

    lax.fori_loop(0, ECH, _zrow, 0)
    _load_idx(0)
    zbase = sid * ROWS_PT
    for k in range(ROWS_PT // ECH):
        pltpu.sync_copy(rows.at[0], aggsh.at[pl.ds(zbase + k * ECH, ECH)])
    plsc.subcore_barrier()

    def _gwait(b):
        pltpu.make_async_copy(x_hbm.at[pl.ds(0, ECH)], rows.at[b],
                              gsem.at[b]).wait()

    def _swait(b):
        pltpu.make_async_copy(x_hbm.at[pl.ds(0, ECH)], rows.at[b],
                              ssem.at[b]).wait()

    for q in range(4):
        if q:
            _load_idx(q)
        for b in range(3):
            pltpu.async_copy(x_hbm.at[sidx.at[b]], rows.at[b], gsem.at[b])

        def _grp(g, carry):
            for b in range(NBUF):
                c = g * NBUF + b
                _gwait(b)
                pltpu.async_copy(rows.at[b], aggsh.at[didx.at[c]],
                                 ssem.at[b], add=True)
                br = (b - 1) % NBUF

                @pl.when(c >= 1)
                def _():
                    _swait(br)

                @pl.when(c + 3 < QCH)
                def _():
                    pltpu.async_copy(x_hbm.at[sidx.at[c + 3]], rows.at[br],
                                     gsem.at[br])
            return carry

        lax.fori_loop(0, QCH // NBUF, _grp, 0)
        _swait((QCH - 1) % NBUF)
    plsc.subcore_barrier()

    obase = sid * ROWS_PT
    pltpu.sync_copy(aggsh.at[pl.ds(obase, ROWS_PT)],
                    out_hbm.at[cid, pl.ds(obase, ROWS_PT)])


def _make_layer(k_halves):
    def body(x_ref, z_ref, w0_ref, w1_ref, b_ref, o_ref):
        tot = jnp.zeros((1024, HID), jnp.float32)
        for k in range(k_halves):
            tot += jnp.dot(x_ref[k], w0_ref[k],
                           preferred_element_type=jnp.float32)
            tot += jnp.dot(z_ref[k, 0] + z_ref[k, 1], w1_ref[k],
                           preferred_element_type=jnp.float32)
        o_ref[...] = tot + b_ref[...]

    return pl.pallas_call(
        body,
        grid=(10,),
        in_specs=[
            pl.BlockSpec((k_halves, 1024, HID), lambda g: (0, g, 0)),
            pl.BlockSpec((k_halves, 2, 1024, HID), lambda g: (0, 0, g, 0)),
            pl.BlockSpec((k_halves, HID, HID), lambda g: (0, 0, 0)),
            pl.BlockSpec((k_halves, HID, HID), lambda g: (0, 0, 0)),
            pl.BlockSpec((1, HID), lambda g: (0, 0)),
        ],
        out_specs=pl.BlockSpec((1024, HID), lambda g: (g, 0)),
        out_shape=jax.ShapeDtypeStruct((NPAD, HID), jnp.float32),
    )


_layer1 = _make_layer(1)
_layer2 = _make_layer(2)


def _msg2_body(x_ref, w1_ref, o_ref):
    tot = jnp.dot(x_ref[0], w1_ref[0], preferred_element_type=jnp.float32)
    tot += jnp.dot(x_ref[1], w1_ref[1], preferred_element_type=jnp.float32)
    o_ref[...] = tot


_msg2 = pl.pallas_call(
    _msg2_body,
    grid=(10,),
    in_specs=[
        pl.BlockSpec((2, 1024, HID), lambda g: (0, g, 0)),
        pl.BlockSpec((2, HID, HID), lambda g: (0, 0, 0)),
    ],
    out_specs=pl.BlockSpec((1024, HID), lambda g: (g, 0)),
    out_shape=jax.ShapeDtypeStruct((NPAD, HID), jnp.float32),
)


def _layerd2_body(x_ref, z_ref, w0_ref, b_ref, o_ref):
    tot = jnp.dot(x_ref[0], w0_ref[0], preferred_element_type=jnp.float32)
    tot += jnp.dot(x_ref[1], w0_ref[1], preferred_element_type=jnp.float32)
    o_ref[...] = tot + z_ref[0] + z_ref[1] + b_ref[...]


_layerd2 = pl.pallas_call(
    _layerd2_body,
    grid=(10,),
    in_specs=[
        pl.BlockSpec((2, 1024, HID), lambda g: (0, g, 0)),
        pl.BlockSpec((2, 1024, HID), lambda g: (0, g, 0)),
        pl.BlockSpec((2, HID, HID), lambda g: (0, 0, 0)),
        pl.BlockSpec((1, HID), lambda g: (0, 0)),
    ],
    out_specs=pl.BlockSpec((1024, HID), lambda g: (g, 0)),
    out_shape=jax.ShapeDtypeStruct((NPAD, HID), jnp.float32),
)


def kernel(conv64, conv128, conv256, conv512, vertices, edges, params):
    del conv64, conv128, conv512

    table = conv256[0].reshape(C3, HW3 * HW3).T
    vx = jnp.pad(vertices[0, :, 0], (0, NPAD - N)).reshape(80, 128)
    vy = jnp.pad(vertices[0, :, 1], (0, NPAD - N)).reshape(80, 128)
    srcp = jnp.pad(edges[0], (0, EPAD - E)).reshape(NTILES, NCH, ECH)
    dstp = jnp.pad(edges[1], (0, EPAD - E),
                   constant_values=N).reshape(NTILES, NCH, ECH)

    pf = params["b3_first"]
    ph = params["b3_hidden"]
    plast = params["b3_last"]

    idx4, w4 = _prep(vx, vy)
    rows4 = _vgather(table, idx4.reshape(4, NPAD))
    v = _interp(rows4.reshape(4, 80, 128, C3), w4).reshape(NPAD, C3)

    vh = jnp.stack([v[:, :HID], v[:, HID:]])
    msg = _msg2(vh, pf["W1"].reshape(2, HID, HID))
    z = _agg(msg, srcp, dstp)
    x = _layerd2(vh, z, pf["W0"].reshape(2, HID, HID), pf["b"][None])

    for i in range(12):
        z = _agg(x, srcp, dstp)
        x = _layer1(x[None], z[None], ph["W0"][i][None], ph["W1"][i][None],
                    ph["b"][i][None])

    z = _agg(x, srcp, dstp)
    w0l = jnp.zeros((HID, HID), jnp.float32).at[:, :3].set(plast["W0"])
    w1l = jnp.zeros((HID, HID), jnp.float32).at[:, :3].set(plast["W1"])
    bl = jnp.zeros((HID,), jnp.float32).at[:3].set(plast["b"])
    out = _layer1(x[None], z[None], w0l[None], w1l[None], bl[None])
    return out[:N, :3]


# speedup vs baseline: 1.0080x; 1.0080x over previous
import functools

import jax
import jax.numpy as jnp
from jax import lax
from jax.experimental import pallas as pl
from jax.experimental.pallas import tpu as pltpu
from jax.experimental.pallas import tpu_sc as plsc

N = 10000
NPAD = 10240
E = 320000
EPAD = 327680
HID = 128
C3 = 256
HW3 = 14
NTILES = 32
VPT = NPAD // NTILES
EPT = EPAD // NTILES
ECH = 128
NCH = EPT // ECH
ROWS_PT = NPAD // 16

_mesh = plsc.VectorSubcoreMesh(core_axis_name="c", subcore_axis_name="s",
                               num_cores=2, num_subcores=16)


def _prep_body(vx_ref, vy_ref, idx_ref, w_ref):
    m = float(HW3 - 1)
    px = (vx_ref[...] + 1.0) * 0.5 * m
    py = (vy_ref[...] + 1.0) * 0.5 * m
    x0f = jnp.floor(px)
    y0f = jnp.floor(py)
    wx = px - x0f
    wy = py - y0f
    x0 = jnp.clip(x0f, 0.0, m).astype(jnp.int32)
    x1 = jnp.clip(x0f + 1.0, 0.0, m).astype(jnp.int32)
    y0 = jnp.clip(y0f, 0.0, m).astype(jnp.int32)
    y1 = jnp.clip(y0f + 1.0, 0.0, m).astype(jnp.int32)
    idx_ref[0] = y0 * HW3 + x0
    idx_ref[1] = y0 * HW3 + x1
    idx_ref[2] = y1 * HW3 + x0
    idx_ref[3] = y1 * HW3 + x1
    w_ref[0] = (1.0 - wx) * (1.0 - wy)
    w_ref[1] = wx * (1.0 - wy)
    w_ref[2] = (1.0 - wx) * wy
    w_ref[3] = wx * wy


_prep = pl.pallas_call(
    _prep_body,
    out_shape=(
        jax.ShapeDtypeStruct((4, 80, 128), jnp.int32),
        jax.ShapeDtypeStruct((4, 80, 128), jnp.float32),
    ),
)


@functools.partial(
    pl.kernel,
    out_type=jax.ShapeDtypeStruct((4, NPAD, C3), jnp.float32),
    mesh=_mesh,
    scratch_types=[
        pltpu.VMEM((64,), jnp.int32),
        pltpu.VMEM((64, C3), jnp.float32),
        pltpu.SemaphoreType.DMA,
    ],
)
def _vgather(tab_hbm, idx_hbm, out_hbm, idxbuf, rowsbuf, sem):
    cid = lax.axis_index("c")
    sid = lax.axis_index("s")
    wid = cid * 16 + sid
    base = wid * VPT
    for q in range(4):
        for t in range(VPT // 64):
            off = base + t * 64
            pltpu.sync_copy(idx_hbm.at[q, pl.ds(off, 64)], idxbuf)
            pltpu.async_copy(tab_hbm.at[idxbuf], rowsbuf, sem).wait()
            pltpu.sync_copy(rowsbuf, out_hbm.at[q, pl.ds(off, 64)])


def _interp_body(r_ref, w_ref, o_ref):
    acc = r_ref[0] * w_ref[0][:, :, None]
    for q in range(1, 4):
        acc = acc + r_ref[q] * w_ref[q][:, :, None]
    o_ref[...] = acc


_interp = pl.pallas_call(
    _interp_body,
    grid=(10,),
    in_specs=[
        pl.BlockSpec((4, 8, 128, C3), lambda g: (0, g, 0, 0)),
        pl.BlockSpec((4, 8, 128), lambda g: (0, g, 0)),
    ],
    out_specs=pl.BlockSpec((8, 128, C3), lambda g: (g, 0, 0)),
    out_shape=jax.ShapeDtypeStruct((80, 128, C3), jnp.float32),
)


NBUF = 2
QCH = NCH // 2


@functools.partial(
    pl.kernel,
    out_type=jax.ShapeDtypeStruct((2, NPAD, HID), jnp.float32),
    mesh=_mesh,
    scratch_types=[
        pltpu.VMEM((QCH, ECH), jnp.int32),
        pltpu.VMEM((QCH, ECH), jnp.int32),
        pltpu.VMEM((NBUF, ECH, HID), jnp.float32),
        pltpu.VMEM_SHARED((NPAD, HID), jnp.float32),
        pltpu.SemaphoreType.DMA((NBUF,)),
        pltpu.SemaphoreType.DMA,
    ],
)
def _agg(x_hbm, src_hbm, dst_hbm, out_hbm, sidx, didx, rows, aggsh,
         gsem, isem):
    cid = lax.axis_index("c")
    sid = lax.axis_index("s")
    wid = cid * 16 + sid

    def _load_idx(q):
        icp = pltpu.async_copy(src_hbm.at[wid, pl.ds(q * QCH, QCH)], sidx,
                               isem)
        icp2 = pltpu.async_copy(dst_hbm.at[wid, pl.ds(q * QCH, QCH)], didx,
                                isem)
        icp.wait()
        icp2.wait()

    zv = jnp.zeros((16,), jnp.float32)

    def _zrow(i, carry):
        for j in range(HID // 16):
            rows[0, i, pl.ds(j * 16, 16)] = zv
        return carry

    lax.fori_loop(0, ECH, _zrow, 0)
    _load_idx(0)
    zbase = sid * ROWS_PT
    for k in range(ROWS_PT // ECH):
        pltpu.sync_copy(rows.at[0], aggsh.at[pl.ds(zbase + k * ECH, ECH)])
    plsc.subcore_barrier()

    def _gwait(b):
        pltpu.make_async_copy(x_hbm.at[pl.ds(0, ECH)], rows.at[b],
                              gsem.at[b]).wait()

    for q in range(2):
        if q:
            _load_idx(q)
        for b in range(NBUF):
            pltpu.async_copy(x_hbm.at[sidx.at[b]], rows.at[b], gsem.at[b])

        def _grp(g, carry):
            for b in range(NBUF):
                c = g * NBUF + b
                _gwait(b)
                pltpu.sync_copy(rows.at[b], aggsh.at[didx.at[c]], add=True)

                @pl.when(c + NBUF < QCH)
                def _():
                    pltpu.async_copy(x_hbm.at[sidx.at[c + NBUF]], rows.at[b],
                                     gsem.at[b])
            return carry

        lax.fori_loop(0, QCH // NBUF, _grp, 0)
    plsc.subcore_barrier()

    obase = sid * ROWS_PT
    pltpu.sync_copy(aggsh.at[pl.ds(obase, ROWS_PT)],
                    out_hbm.at[cid, pl.ds(obase, ROWS_PT)])


def _make_layer(k_halves):
    def body(x_ref, z_ref, w0_ref, w1_ref, b_ref, o_ref):
        tot = jnp.zeros((1024, HID), jnp.float32)
        for k in range(k_halves):
            tot += jnp.dot(x_ref[k], w0_ref[k],
                           preferred_element_type=jnp.float32)
            tot += jnp.dot(z_ref[k, 0] + z_ref[k, 1], w1_ref[k],
                           preferred_element_type=jnp.float32)
        o_ref[...] = tot + b_ref[...]

    return pl.pallas_call(
        body,
        grid=(10,),
        in_specs=[
            pl.BlockSpec((k_halves, 1024, HID), lambda g: (0, g, 0)),
            pl.BlockSpec((k_halves, 2, 1024, HID), lambda g: (0, 0, g, 0)),
            pl.BlockSpec((k_halves, HID, HID), lambda g: (0, 0, 0)),
            pl.BlockSpec((k_halves, HID, HID), lambda g: (0, 0, 0)),
            pl.BlockSpec((1, HID), lambda g: (0, 0)),
        ],
        out_specs=pl.BlockSpec((1024, HID), lambda g: (g, 0)),
        out_shape=jax.ShapeDtypeStruct((NPAD, HID), jnp.float32),
    )


_layer1 = _make_layer(1)
_layer2 = _make_layer(2)


def _msg2_body(x_ref, w1_ref, o_ref):
    tot = jnp.dot(x_ref[0], w1_ref[0], preferred_element_type=jnp.float32)
    tot += jnp.dot(x_ref[1], w1_ref[1], preferred_element_type=jnp.float32)
    o_ref[...] = tot


_msg2 = pl.pallas_call(
    _msg2_body,
    grid=(10,),
    in_specs=[
        pl.BlockSpec((2, 1024, HID), lambda g: (0, g, 0)),
        pl.BlockSpec((2, HID, HID), lambda g: (0, 0, 0)),
    ],
    out_specs=pl.BlockSpec((1024, HID), lambda g: (g, 0)),
    out_shape=jax.ShapeDtypeStruct((NPAD, HID), jnp.float32),
)


def _layerd2_body(x_ref, z_ref, w0_ref, b_ref, o_ref):
    tot = jnp.dot(x_ref[0], w0_ref[0], preferred_element_type=jnp.float32)
    tot += jnp.dot(x_ref[1], w0_ref[1], preferred_element_type=jnp.float32)
    o_ref[...] = tot + z_ref[0] + z_ref[1] + b_ref[...]


_layerd2 = pl.pallas_call(
    _layerd2_body,
    grid=(10,),
    in_specs=[
        pl.BlockSpec((2, 1024, HID), lambda g: (0, g, 0)),
        pl.BlockSpec((2, 1024, HID), lambda g: (0, g, 0)),
        pl.BlockSpec((2, HID, HID), lambda g: (0, 0, 0)),
        pl.BlockSpec((1, HID), lambda g: (0, 0)),
    ],
    out_specs=pl.BlockSpec((1024, HID), lambda g: (g, 0)),
    out_shape=jax.ShapeDtypeStruct((NPAD, HID), jnp.float32),
)


def kernel(conv64, conv128, conv256, conv512, vertices, edges, params):
    del conv64, conv128, conv512

    table = conv256[0].reshape(C3, HW3 * HW3).T
    vx = jnp.pad(vertices[0, :, 0], (0, NPAD - N)).reshape(80, 128)
    vy = jnp.pad(vertices[0, :, 1], (0, NPAD - N)).reshape(80, 128)
    srcp = jnp.pad(edges[0], (0, EPAD - E)).reshape(NTILES, NCH, ECH)
    dstp = jnp.pad(edges[1], (0, EPAD - E),
                   constant_values=N).reshape(NTILES, NCH, ECH)

    pf = params["b3_first"]
    ph = params["b3_hidden"]
    plast = params["b3_last"]

    idx4, w4 = _prep(vx, vy)
    rows4 = _vgather(table, idx4.reshape(4, NPAD))
    v = _interp(rows4.reshape(4, 80, 128, C3), w4).reshape(NPAD, C3)

    vh = jnp.stack([v[:, :HID], v[:, HID:]])
    msg = _msg2(vh, pf["W1"].reshape(2, HID, HID))
    z = _agg(msg, srcp, dstp)
    x = _layerd2(vh, z, pf["W0"].reshape(2, HID, HID), pf["b"][None])

    for i in range(12):
        z = _agg(x, srcp, dstp)
        x = _layer1(x[None], z[None], ph["W0"][i][None], ph["W1"][i][None],
                    ph["b"][i][None])

    z = _agg(x, srcp, dstp)
    w0l = jnp.zeros((HID, HID), jnp.float32).at[:, :3].set(plast["W0"])
    w1l = jnp.zeros((HID, HID), jnp.float32).at[:, :3].set(plast["W1"])
    bl = jnp.zeros((HID,), jnp.float32).at[:3].set(plast["b"])
    out = _layer1(x[None], z[None], w0l[None], w1l[None], bl[None])
    return out[:N, :3]


# speedup vs baseline: 1.0196x; 1.0114x over previous
import functools

import jax
import jax.numpy as jnp
from jax import lax
from jax.experimental import pallas as pl
from jax.experimental.pallas import tpu as pltpu
from jax.experimental.pallas import tpu_sc as plsc

N = 10000
NPAD = 10240
E = 320000
EPAD = 327680
HID = 128
C3 = 256
HW3 = 14
NTILES = 32
VPT = NPAD // NTILES
EPT = EPAD // NTILES
ECH = 128
NCH = EPT // ECH
ROWS_PT = NPAD // 16

_mesh = plsc.VectorSubcoreMesh(core_axis_name="c", subcore_axis_name="s",
                               num_cores=2, num_subcores=16)


def _prep_body(vx_ref, vy_ref, idx_ref, w_ref):
    m = float(HW3 - 1)
    px = (vx_ref[...] + 1.0) * 0.5 * m
    py = (vy_ref[...] + 1.0) * 0.5 * m
    x0f = jnp.floor(px)
    y0f = jnp.floor(py)
    wx = px - x0f
    wy = py - y0f
    x0 = jnp.clip(x0f, 0.0, m).astype(jnp.int32)
    x1 = jnp.clip(x0f + 1.0, 0.0, m).astype(jnp.int32)
    y0 = jnp.clip(y0f, 0.0, m).astype(jnp.int32)
    y1 = jnp.clip(y0f + 1.0, 0.0, m).astype(jnp.int32)
    idx_ref[0] = y0 * HW3 + x0
    idx_ref[1] = y0 * HW3 + x1
    idx_ref[2] = y1 * HW3 + x0
    idx_ref[3] = y1 * HW3 + x1
    w_ref[0] = (1.0 - wx) * (1.0 - wy)
    w_ref[1] = wx * (1.0 - wy)
    w_ref[2] = (1.0 - wx) * wy
    w_ref[3] = wx * wy


_prep = pl.pallas_call(
    _prep_body,
    out_shape=(
        jax.ShapeDtypeStruct((4, 80, 128), jnp.int32),
        jax.ShapeDtypeStruct((4, 80, 128), jnp.float32),
    ),
)


@functools.partial(
    pl.kernel,
    out_type=jax.ShapeDtypeStruct((4, NPAD, C3), jnp.float32),
    mesh=_mesh,
    scratch_types=[
        pltpu.VMEM((64,), jnp.int32),
        pltpu.VMEM((64, C3), jnp.float32),
        pltpu.SemaphoreType.DMA,
    ],
)
def _vgather(tab_hbm, idx_hbm, out_hbm, idxbuf, rowsbuf, sem):
    cid = lax.axis_index("c")
    sid = lax.axis_index("s")
    wid = cid * 16 + sid
    base = wid * VPT
    for q in range(4):
        for t in range(VPT // 64):
            off = base + t * 64
            pltpu.sync_copy(idx_hbm.at[q, pl.ds(off, 64)], idxbuf)
            pltpu.async_copy(tab_hbm.at[idxbuf], rowsbuf, sem).wait()
            pltpu.sync_copy(rowsbuf, out_hbm.at[q, pl.ds(off, 64)])


def _interp_body(r_ref, w_ref, o_ref):
    acc = r_ref[0] * w_ref[0][:, :, None]
    for q in range(1, 4):
        acc = acc + r_ref[q] * w_ref[q][:, :, None]
    o_ref[...] = acc


_interp = pl.pallas_call(
    _interp_body,
    grid=(10,),
    in_specs=[
        pl.BlockSpec((4, 8, 128, C3), lambda g: (0, g, 0, 0)),
        pl.BlockSpec((4, 8, 128), lambda g: (0, g, 0)),
    ],
    out_specs=pl.BlockSpec((8, 128, C3), lambda g: (g, 0, 0)),
    out_shape=jax.ShapeDtypeStruct((80, 128, C3), jnp.float32),
)


NBUF = 2
QCH = NCH // 2


@functools.partial(
    pl.kernel,
    out_type=jax.ShapeDtypeStruct((2, NPAD, HID), jnp.float32),
    mesh=_mesh,
    scratch_types=[
        pltpu.VMEM((QCH, ECH), jnp.int32),
        pltpu.VMEM((QCH, ECH), jnp.int32),
        pltpu.VMEM((NBUF, ECH, HID), jnp.float32),
        pltpu.VMEM_SHARED((NPAD, HID), jnp.float32),
        pltpu.SemaphoreType.DMA((NBUF,)),
        pltpu.SemaphoreType.DMA,
    ],
)
def _agg(x_hbm, src_hbm, dst_hbm, out_hbm, sidx, didx, rows, aggsh,
         gsem, isem):
    cid = lax.axis_index("c")
    sid = lax.axis_index("s")
    wid = cid * 16 + sid

    def _load_idx(q):
        icp = pltpu.async_copy(src_hbm.at[wid, pl.ds(q * QCH, QCH)], sidx,
                               isem)
        icp2 = pltpu.async_copy(dst_hbm.at[wid, pl.ds(q * QCH, QCH)], didx,
                                isem)
        icp.wait()
        icp2.wait()

    zv = jnp.zeros((16,), jnp.float32)

    def _zrow(i, carry):
        for j in range(HID // 16):
            rows[0, i, pl.ds(j * 16, 16)] = zv
        return carry

    lax.fori_loop(0, ECH, _zrow, 0)
    _load_idx(0)
    zbase = sid * ROWS_PT
    for k in range(ROWS_PT // ECH):
        pltpu.sync_copy(rows.at[0], aggsh.at[pl.ds(zbase + k * ECH, ECH)])
    plsc.subcore_barrier()

    def _gwait(b):
        pltpu.make_async_copy(x_hbm.at[pl.ds(0, ECH)], rows.at[b],
                              gsem.at[b]).wait()

    for q in range(2):
        if q:
            _load_idx(q)
        for b in range(NBUF):
            pltpu.async_copy(x_hbm.at[sidx.at[b]], rows.at[b], gsem.at[b])

        def _grp(g, carry):
            for b in range(NBUF):
                c = g * NBUF + b
                _gwait(b)
                pltpu.sync_copy(rows.at[b], aggsh.at[didx.at[c]], add=True)

                @pl.when(c + NBUF < QCH)
                def _():
                    pltpu.async_copy(x_hbm.at[sidx.at[c + NBUF]], rows.at[b],
                                     gsem.at[b])
            return carry

        lax.fori_loop(0, QCH // NBUF, _grp, 0)
    plsc.subcore_barrier()

    obase = sid * ROWS_PT
    pltpu.sync_copy(aggsh.at[pl.ds(obase, ROWS_PT)],
                    out_hbm.at[cid, pl.ds(obase, ROWS_PT)])


def _make_layer(k_halves):
    def body(x_ref, z_ref, w0_ref, w1_ref, b_ref, o_ref):
        tot = jnp.zeros((1024, HID), jnp.float32)
        for k in range(k_halves):
            tot += jnp.dot(x_ref[k], w0_ref[k],
                           preferred_element_type=jnp.float32)
            tot += jnp.dot(z_ref[k, 0] + z_ref[k, 1], w1_ref[k],
                           preferred_element_type=jnp.float32)
        o_ref[...] = tot + b_ref[...]

    return pl.pallas_call(
        body,
        grid=(10,),
        in_specs=[
            pl.BlockSpec((k_halves, 1024, HID), lambda g: (0, g, 0)),
            pl.BlockSpec((k_halves, 2, 1024, HID), lambda g: (0, 0, g, 0)),
            pl.BlockSpec((k_halves, HID, HID), lambda g: (0, 0, 0)),
            pl.BlockSpec((k_halves, HID, HID), lambda g: (0, 0, 0)),
            pl.BlockSpec((1, HID), lambda g: (0, 0)),
        ],
        out_specs=pl.BlockSpec((1024, HID), lambda g: (g, 0)),
        out_shape=jax.ShapeDtypeStruct((NPAD, HID), jnp.float32),
    )


_layer1 = _make_layer(1)
_layer2 = _make_layer(2)


def kernel(conv64, conv128, conv256, conv512, vertices, edges, params):
    del conv64, conv128, conv512

    table = conv256[0].reshape(C3, HW3 * HW3).T
    vx = jnp.pad(vertices[0, :, 0], (0, NPAD - N)).reshape(80, 128)
    vy = jnp.pad(vertices[0, :, 1], (0, NPAD - N)).reshape(80, 128)
    srcp = jnp.pad(edges[0], (0, EPAD - E)).reshape(NTILES, NCH, ECH)
    dstp = jnp.pad(edges[1], (0, EPAD - E),
                   constant_values=N).reshape(NTILES, NCH, ECH)

    pf = params["b3_first"]
    ph = params["b3_hidden"]
    plast = params["b3_last"]

    idx4, w4 = _prep(vx, vy)
    rows4 = _vgather(table, idx4.reshape(4, NPAD))
    v = _interp(rows4.reshape(4, 80, 128, C3), w4).reshape(NPAD, C3)

    vh = jnp.stack([v[:, :HID], v[:, HID:]])
    z0 = _agg(vh[0], srcp, dstp)
    z1 = _agg(vh[1], srcp, dstp)
    x = _layer2(vh, jnp.stack([z0, z1]),
                pf["W0"].reshape(2, HID, HID),
                pf["W1"].reshape(2, HID, HID),
                pf["b"][None])

    for i in range(12):
        z = _agg(x, srcp, dstp)
        x = _layer1(x[None], z[None], ph["W0"][i][None], ph["W1"][i][None],
                    ph["b"][i][None])

    z = _agg(x, srcp, dstp)
    w0l = jnp.zeros((HID, HID), jnp.float32).at[:, :3].set(plast["W0"])
    w1l = jnp.zeros((HID, HID), jnp.float32).at[:, :3].set(plast["W1"])
    bl = jnp.zeros((HID,), jnp.float32).at[:3].set(plast["b"])
    out = _layer1(x[None], z[None], w0l[None], w1l[None], bl[None])
    return out[:N, :3]
